# Initial kernel scaffold; baseline (speedup 1.0000x reference)
import jax
import jax.numpy as jnp
from jax.experimental import pallas as pl


def kernel(input, z, W_grower):
    raise NotImplementedError("write your pallas kernel here")


# speedup vs baseline: 70.0473x; 70.0473x over previous
import jax
import jax.numpy as jnp
from jax import lax
from jax.experimental import pallas as pl
from jax.experimental.pallas import tpu as pltpu
from jax.experimental.pallas import tpu_sc as plsc

LEVEL_RES = (16, 32, 64)
N_FEATS = 4
B = 16
N_PTS = 65536
N_CELLS = sum(r ** 3 for r in LEVEL_RES)
N_PARAMS = N_CELLS * N_FEATS
CELL_OFF = (0, 16 ** 3, 16 ** 3 + 32 ** 3)
N_OUT = 3 * N_FEATS

NC, NS = 2, 16
N_WORKERS = NC * NS
PTS_PER_WORKER = B * N_PTS // N_WORKERS
BLK = 128
N_BLKS = PTS_PER_WORKER // BLK
N_GRP = BLK // 16


def _mm_body(z_ref, w_ref, o_ref):
    o_ref[...] = jnp.dot(z_ref[...], w_ref[...],
                         preferred_element_type=jnp.float32)


def _grow(z, W_grower):
    BN = 8192
    return pl.pallas_call(
        _mm_body,
        grid=(N_PARAMS // BN,),
        in_specs=[
            pl.BlockSpec((B, z.shape[1]), lambda i: (0, 0)),
            pl.BlockSpec((z.shape[1], BN), lambda i: (0, i)),
        ],
        out_specs=pl.BlockSpec((B, BN), lambda i: (0, i)),
        out_shape=jax.ShapeDtypeStruct((B, N_PARAMS), jnp.float32),
        compiler_params=pltpu.CompilerParams(
            dimension_semantics=("arbitrary",)),
    )(z, W_grower)


def _interp_body(pts_hbm, lodf_hbm, lod8_hbm, out_hbm,
                 grid0, pts_v, out_v,
                 idx1, w1, p1, idx2, w2, p2, rows1, rows2, sem):
    c = lax.axis_index("c")
    s = lax.axis_index("s")
    wid = s * NC + c
    b = wid // 2
    base_pt = (wid % 2) * PTS_PER_WORKER

    pltpu.sync_copy(lodf_hbm.at[b, pl.ds(0, CELL_OFF[1] * N_FEATS)], grid0)

    iota = lax.iota(jnp.int32, 16)

    def block(i, carry):
        start = base_pt + i * BLK
        pltpu.sync_copy(pts_hbm.at[b, pl.ds(start * 3, BLK * 3)], pts_v)

        for g in range(N_GRP):
            pvec = iota + (g * 16)
            pv3 = pvec * 3
            co = []
            for d in range(3):
                xr = plsc.load_gather(pts_v, [pv3 + d])
                co.append(jnp.clip(xr * 0.5 + 0.5, 0.0, 1.0))
            pv12 = pvec * 12
            for lvl, R in enumerate(LEVEL_RES):
                x0i, w, u = [], [], []
                for d in range(3):
                    xs = co[d] * float(R - 1)
                    xi = jnp.clip(xs.astype(jnp.int32), 0, R - 2)
                    wd = jnp.clip(xs - xi.astype(jnp.float32), 0.0, 1.0)
                    x0i.append(xi)
                    w.append(wd)
                    u.append(1.0 - wd)
                basei = (x0i[0] * R + x0i[1]) * R + x0i[2] + CELL_OFF[lvl]
                wxy = [(w[0] if dx else u[0]) * (w[1] if dy else u[1])
                       for dx in (0, 1) for dy in (0, 1)]
                if lvl == 0:
                    base4 = basei * 4
                    acc = [jnp.zeros((16,), jnp.float32) for _ in range(4)]
                    for ci, (dx, dy, dz) in enumerate(
                            (dx, dy, dz) for dx in (0, 1) for dy in (0, 1)
                            for dz in (0, 1)):
                        wc = wxy[dx * 2 + dy] * (w[2] if dz else u[2])
                        idx4 = base4 + ((dx * R * R + dy * R + dz) * 4)
                        for f in range(4):
                            v = plsc.load_gather(grid0, [idx4 + f])
                            acc[f] = acc[f] + v * wc
                    for f in range(4):
                        plsc.store_scatter(out_v, [pv12 + f], acc[f])
                else:
                    iref, wref, pref = ((idx1, w1, p1) if lvl == 1
                                        else (idx2, w2, p2))
                    for j, (dx, dy) in enumerate(
                            (dx, dy) for dx in (0, 1) for dy in (0, 1)):
                        c0 = basei + (dx * R * R + dy * R)
                        r0 = jnp.right_shift(c0, 1)
                        iref[2 * j, pl.ds(g * 16, 16)] = r0
                        iref[2 * j + 1, pl.ds(g * 16, 16)] = r0 + 1
                        pref[pl.ds(j * BLK + g * 16, 16)] = (
                            jnp.left_shift(jnp.bitwise_and(c0, 1), 2))
                        for dz in (0, 1):
                            wc = wxy[dx * 2 + dy] * (w[2] if dz else u[2])
                            wref[pl.ds((2 * j + dz) * BLK + g * 16, 16)] = wc

        cps = []
        for sl in range(8):
            cps.append(pltpu.async_copy(
                lod8_hbm.at[b].at[idx1.at[sl]], rows1.at[sl], sem))
            cps.append(pltpu.async_copy(
                lod8_hbm.at[b].at[idx2.at[sl]], rows2.at[sl], sem))
        for cp in cps:
            cp.wait()

        for g in range(N_GRP):
            pvec = iota + (g * 16)
            pv12 = pvec * 12
            for lvl, rows, wref, pref in ((1, rows1, w1, p1),
                                          (2, rows2, w2, p2)):
                acc = [jnp.zeros((16,), jnp.float32) for _ in range(4)]
                for j in range(4):
                    par4 = pref[pl.ds(j * BLK + g * 16, 16)]
                    for dz in (0, 1):
                        wc = wref[pl.ds((2 * j + dz) * BLK + g * 16, 16)]
                        t0 = par4 + (4 * dz)
                        slotv = jnp.right_shift(t0, 3) + (2 * j)
                        mb = jnp.bitwise_and(t0, 7)
                        for f in range(4):
                            v = plsc.load_gather(rows, [slotv, pvec, mb + f])
                            acc[f] = acc[f] + v * wc
                for f in range(4):
                    plsc.store_scatter(out_v, [pv12 + (4 * lvl + f)], acc[f])

        pltpu.sync_copy(out_v, out_hbm.at[b, pl.ds(start * N_OUT,
                                                   BLK * N_OUT)])
        return carry

    lax.fori_loop(0, N_BLKS, block, 0)


def _interp(pts_flat, lod_flat, lod8):
    mesh = plsc.VectorSubcoreMesh(core_axis_name="c", subcore_axis_name="s")
    f = pl.kernel(
        _interp_body,
        out_type=jax.ShapeDtypeStruct((B, N_PTS * N_OUT), jnp.float32),
        mesh=mesh,
        scratch_types=[
            pltpu.VMEM((CELL_OFF[1] * N_FEATS,), jnp.float32),
            pltpu.VMEM((BLK * 3,), jnp.float32),
            pltpu.VMEM((BLK * N_OUT,), jnp.float32),
            pltpu.VMEM((8, BLK), jnp.int32),
            pltpu.VMEM((8 * BLK,), jnp.float32),
            pltpu.VMEM((4 * BLK,), jnp.int32),
            pltpu.VMEM((8, BLK), jnp.int32),
            pltpu.VMEM((8 * BLK,), jnp.float32),
            pltpu.VMEM((4 * BLK,), jnp.int32),
            pltpu.VMEM((8, BLK, 8), jnp.float32),
            pltpu.VMEM((8, BLK, 8), jnp.float32),
            pltpu.SemaphoreType.DMA,
        ],
        compiler_params=pltpu.CompilerParams(
            needs_layout_passes=False, use_tc_tiling_on_sc=False),
    )
    return f(pts_flat, lod_flat, lod8)


@jax.jit
def kernel(input, z, W_grower):
    lod = _grow(z, W_grower)
    out = _interp(input.reshape(B, N_PTS * 3), lod,
                  lod.reshape(B, N_PARAMS // 8, 8))
    return out.reshape(B, N_PTS, N_OUT)
